# baseline (speedup 1.0000x reference)
import functools

import jax
import jax.numpy as jnp
from jax import lax
from jax.experimental import pallas as pl
from jax.experimental.pallas import tpu as pltpu
from jax.experimental.pallas import tpu_sc as plsc

B = 1024
IB = 1024
N = 4096
K = 12
NA = 4096
PACK = 32
NWORDS = NA // PACK


_NBLK = 512


def _addr_body(bits_ref, conn_ref, addr_ref):
    conn = conn_ref[...]
    ii = lax.broadcasted_iota(jnp.int32, (_NBLK, IB), 1)
    wt_hi = jnp.zeros((_NBLK, IB), jnp.float32)
    wt_lo = jnp.zeros((_NBLK, IB), jnp.float32)
    for k in range(K // 2):
        w = float(2 ** (K - 1 - k))
        wt_hi = wt_hi + jnp.where(conn[:, k : k + 1] == ii, w, 0.0)
    for k in range(K // 2, K):
        w = float(2 ** (K - 1 - k))
        wt_lo = wt_lo + jnp.where(conn[:, k : k + 1] == ii, w, 0.0)
    bits = bits_ref[...].astype(jnp.bfloat16)
    nt = (((1,), (1,)), ((), ()))
    addr = lax.dot_general(
        bits, wt_hi.astype(jnp.bfloat16), nt,
        preferred_element_type=jnp.float32,
    ) + lax.dot_general(
        bits, wt_lo.astype(jnp.bfloat16), nt,
        preferred_element_type=jnp.float32,
    )
    addr_ref[...] = addr.astype(jnp.int32).astype(jnp.int16)


def _addresses(bits_u8, connections):
    return pl.pallas_call(
        _addr_body,
        grid=(N // _NBLK,),
        in_specs=[
            pl.BlockSpec((B, IB), lambda i: (0, 0)),
            pl.BlockSpec((_NBLK, K), lambda i: (i, 0)),
        ],
        out_specs=pl.BlockSpec((B, _NBLK), lambda i: (0, i)),
        out_shape=jax.ShapeDtypeStruct((B, N), jnp.int16),
    )(bits_u8, connections)


_MBLK = 512


def _pack_body(mem_ref, packed_ref):
    m = (mem_ref[...] == 1).astype(jnp.bfloat16)
    a = lax.broadcasted_iota(jnp.int32, (NA, NWORDS), 0)
    w = lax.broadcasted_iota(jnp.int32, (NA, NWORDS), 1)
    sub = a % PACK
    blk = (a // PACK) == w
    plo = jnp.where(blk & (sub < 16), 1 << (sub & 15), 0).astype(jnp.bfloat16)
    phi = jnp.where(blk & (sub >= 16), 1 << (sub & 15), 0).astype(jnp.bfloat16)
    nn = (((1,), (0,)), ((), ()))
    lo = lax.dot_general(m, plo, nn, preferred_element_type=jnp.float32)
    hi = lax.dot_general(m, phi, nn, preferred_element_type=jnp.float32)
    packed_ref[...] = lo.astype(jnp.int32) | (hi.astype(jnp.int32) << 16)


def _pack_memory(memory):
    return pl.pallas_call(
        _pack_body,
        grid=(N // _MBLK,),
        in_specs=[pl.BlockSpec((_MBLK, NA), lambda i: (i, 0))],
        out_specs=pl.BlockSpec((_MBLK, NWORDS), lambda i: (i, 0)),
        out_shape=jax.ShapeDtypeStruct((N, NWORDS), jnp.int32),
    )(memory)


_NTILES = 32
_NPT = N // _NTILES
_CB = 128
_LANES = 16


def _lookup_body(addr_hbm, packed_hbm, out_hbm,
                 tab_v, a0, a1, o0, o1, sa0, sa1, so0, so1):
    cid = lax.axis_index("c")
    sid = lax.axis_index("s")
    wid = sid * 2 + cid
    n0 = wid * _NPT

    pltpu.sync_copy(packed_hbm.at[pl.ds(n0, _NPT), :], tab_v)

    lane = lax.iota(jnp.int32, _LANES)
    ngroups = _NPT // (2 * _LANES)
    nvec_e = [lane * 2 + g * 2 * _LANES for g in range(ngroups)]
    nvec_o = [lane * 2 + (g * 2 * _LANES + 1) for g in range(ngroups)]

    abuf, obuf = (a0, a1), (o0, o1)
    asem, osem = (sa0, sa1), (so0, so1)
    nchunks = B // _CB

    def start_in(c):
        return pltpu.async_copy(
            addr_hbm.at[pl.ds(c * _CB, _CB), pl.ds(n0, _NPT)],
            abuf[c % 2], asem[c % 2])

    in_cps = [None] * nchunks
    out_cps = [None] * nchunks
    in_cps[0] = start_in(0)
    for c in range(nchunks):
        av, ov = abuf[c % 2], obuf[c % 2]
        if c + 1 < nchunks:
            in_cps[c + 1] = start_in(c + 1)
        in_cps[c].wait()
        if c >= 2:
            out_cps[c - 2].wait()

        @plsc.parallel_loop(0, _CB, unroll=4)
        def _row(r):
            for g in range(ngroups):
                a2 = av[r, pl.ds(g * 2 * _LANES, 2 * _LANES)]
                w2 = plsc.bitcast(a2, jnp.int32)
                ae = jnp.bitwise_and(w2, 0xFFF)
                ao = lax.shift_right_logical(w2, 16)
                we = plsc.load_gather(
                    tab_v, [nvec_e[g], lax.shift_right_logical(ae, 5)])
                wo = plsc.load_gather(
                    tab_v, [nvec_o[g], lax.shift_right_logical(ao, 5)])
                be = jnp.bitwise_and(
                    lax.shift_right_logical(we, jnp.bitwise_and(ae, 31)), 1)
                bo = jnp.bitwise_and(
                    lax.shift_right_logical(wo, jnp.bitwise_and(ao, 31)), 1)
                comb = jnp.bitwise_or(be, lax.shift_left(bo, 16))
                ov[r, pl.ds(g * 2 * _LANES, 2 * _LANES)] = plsc.bitcast(
                    comb, jnp.int16)

        out_cps[c] = pltpu.async_copy(
            ov, out_hbm.at[pl.ds(c * _CB, _CB), pl.ds(n0, _NPT)], osem[c % 2])
    out_cps[-2].wait()
    out_cps[-1].wait()


def _lookup(addresses, packed):
    mesh = plsc.VectorSubcoreMesh(core_axis_name="c", subcore_axis_name="s")
    f = pl.kernel(
        _lookup_body,
        out_type=jax.ShapeDtypeStruct((B, N), jnp.int16),
        mesh=mesh,
        compiler_params=pltpu.CompilerParams(
            use_tc_tiling_on_sc=False, needs_layout_passes=False
        ),
        scratch_types=[
            pltpu.VMEM((_NPT, NWORDS), jnp.int32),
            pltpu.VMEM((_CB, _NPT), jnp.int16),
            pltpu.VMEM((_CB, _NPT), jnp.int16),
            pltpu.VMEM((_CB, _NPT), jnp.int16),
            pltpu.VMEM((_CB, _NPT), jnp.int16),
            pltpu.SemaphoreType.DMA,
            pltpu.SemaphoreType.DMA,
            pltpu.SemaphoreType.DMA,
            pltpu.SemaphoreType.DMA,
        ],
    )
    return f(addresses, packed)


def kernel(input_bits, connections, memory):
    bits_u8 = input_bits.astype(jnp.uint8)
    addresses = _addresses(bits_u8, connections)
    packed = _pack_memory(memory)
    out = _lookup(addresses, packed)
    return out.astype(jnp.bool_)


# speedup vs baseline: 1.4276x; 1.4276x over previous
import functools

import jax
import jax.numpy as jnp
from jax import lax
from jax.experimental import pallas as pl
from jax.experimental.pallas import tpu as pltpu
from jax.experimental.pallas import tpu_sc as plsc

B = 1024
IB = 1024
N = 4096
K = 12
NA = 4096
PACK = 32
NWORDS = NA // PACK


_NBLK = 512


def _addr_body(bits_ref, conn_ref, addr_ref):
    conn = conn_ref[...]
    ii = lax.broadcasted_iota(jnp.int32, (_NBLK, IB), 1)
    wt_hi = jnp.zeros((_NBLK, IB), jnp.float32)
    wt_lo = jnp.zeros((_NBLK, IB), jnp.float32)
    for k in range(K // 2):
        w = float(2 ** (K - 1 - k))
        wt_hi = wt_hi + jnp.where(conn[:, k : k + 1] == ii, w, 0.0)
    for k in range(K // 2, K):
        w = float(2 ** (K - 1 - k))
        wt_lo = wt_lo + jnp.where(conn[:, k : k + 1] == ii, w, 0.0)
    bits = bits_ref[...].astype(jnp.bfloat16)
    nt = (((1,), (1,)), ((), ()))
    addr = lax.dot_general(
        bits, wt_hi.astype(jnp.bfloat16), nt,
        preferred_element_type=jnp.float32,
    ) + lax.dot_general(
        bits, wt_lo.astype(jnp.bfloat16), nt,
        preferred_element_type=jnp.float32,
    )
    ai = addr.astype(jnp.int32)
    pairs = []
    for sub in range(_NBLK // 128):
        lo = lax.slice(ai, (0, sub * 128), (B, sub * 128 + 64))
        hi = lax.slice(ai, (0, sub * 128 + 64), (B, sub * 128 + 128))
        pairs.append(jnp.bitwise_or(lo, lax.shift_left(hi, 16)))
    addr_ref[...] = jnp.concatenate(pairs, axis=1)


def _addresses(bits_u8, connections):
    return pl.pallas_call(
        _addr_body,
        grid=(N // _NBLK,),
        in_specs=[
            pl.BlockSpec((B, IB), lambda i: (0, 0)),
            pl.BlockSpec((_NBLK, K), lambda i: (i, 0)),
        ],
        out_specs=pl.BlockSpec((B, _NBLK // 2), lambda i: (0, i)),
        out_shape=jax.ShapeDtypeStruct((B, N // 2), jnp.int32),
    )(bits_u8, connections)


_MBLK = 512


def _pack_body(mem_ref, packed_ref):
    m = (mem_ref[...] == 1).astype(jnp.bfloat16)
    a = lax.broadcasted_iota(jnp.int32, (NA, NWORDS), 0)
    w = lax.broadcasted_iota(jnp.int32, (NA, NWORDS), 1)
    sub = a % PACK
    blk = (a // PACK) == w
    plo = jnp.where(blk & (sub < 16), 1 << (sub & 15), 0).astype(jnp.bfloat16)
    phi = jnp.where(blk & (sub >= 16), 1 << (sub & 15), 0).astype(jnp.bfloat16)
    nn = (((1,), (0,)), ((), ()))
    lo = lax.dot_general(m, plo, nn, preferred_element_type=jnp.float32)
    hi = lax.dot_general(m, phi, nn, preferred_element_type=jnp.float32)
    packed_ref[...] = lo.astype(jnp.int32) | (hi.astype(jnp.int32) << 16)


def _pack_memory(memory):
    return pl.pallas_call(
        _pack_body,
        grid=(N // _MBLK,),
        in_specs=[pl.BlockSpec((_MBLK, NA), lambda i: (i, 0))],
        out_specs=pl.BlockSpec((_MBLK, NWORDS), lambda i: (i, 0)),
        out_shape=jax.ShapeDtypeStruct((N, NWORDS), jnp.int32),
    )(memory)


_NTILES = 32
_NPT = N // _NTILES
_NWT = _NPT // 2
_CB = 256
_LANES = 16


def _lookup_body(addr_hbm, packed_hbm, out_hbm,
                 tab_v, a0, a1, o0, o1, sa0, sa1, so0, so1):
    cid = lax.axis_index("c")
    sid = lax.axis_index("s")
    wid = sid * 2 + cid
    n0 = wid * _NPT
    w0 = wid * _NWT

    pltpu.sync_copy(packed_hbm.at[pl.ds(n0, _NPT), :], tab_v)

    lane = lax.iota(jnp.int32, _LANES)
    nvec = [lane + g * _LANES for g in range(_NPT // _LANES)]

    abuf, obuf = (a0, a1), (o0, o1)
    asem, osem = (sa0, sa1), (so0, so1)
    nchunks = B // _CB

    def start_in(c):
        return pltpu.async_copy(
            addr_hbm.at[pl.ds(c * _CB, _CB), pl.ds(w0, _NWT)],
            abuf[c % 2], asem[c % 2])

    in_cps = [None] * nchunks
    out_cps = [None] * nchunks
    in_cps[0] = start_in(0)
    for c in range(nchunks):
        av, ov = abuf[c % 2], obuf[c % 2]
        if c + 1 < nchunks:
            in_cps[c + 1] = start_in(c + 1)
        in_cps[c].wait()
        if c >= 2:
            out_cps[c - 2].wait()

        @plsc.parallel_loop(0, _CB, unroll=4)
        def _row(r):
            for gp in range(_NWT // _LANES):
                w = av[r, pl.ds(gp * _LANES, _LANES)]
                for half in range(2):
                    if half == 0:
                        a = jnp.bitwise_and(w, 0xFFF)
                    else:
                        a = lax.shift_right_logical(w, 16)
                    g = gp + 4 * half
                    word = plsc.load_gather(
                        tab_v, [nvec[g], lax.shift_right_logical(a, 5)])
                    bit = jnp.bitwise_and(
                        lax.shift_right_logical(
                            word, jnp.bitwise_and(a, 31)), 1)
                    ov[r, pl.ds(g * _LANES, _LANES)] = bit

        out_cps[c] = pltpu.async_copy(
            ov, out_hbm.at[pl.ds(c * _CB, _CB), pl.ds(n0, _NPT)], osem[c % 2])
    out_cps[-2].wait()
    out_cps[-1].wait()


def _lookup(addresses, packed):
    mesh = plsc.VectorSubcoreMesh(core_axis_name="c", subcore_axis_name="s")
    f = pl.kernel(
        _lookup_body,
        out_type=jax.ShapeDtypeStruct((B, N), jnp.int32),
        mesh=mesh,
        compiler_params=pltpu.CompilerParams(
            use_tc_tiling_on_sc=False, needs_layout_passes=False
        ),
        scratch_types=[
            pltpu.VMEM((_NPT, NWORDS), jnp.int32),
            pltpu.VMEM((_CB, _NWT), jnp.int32),
            pltpu.VMEM((_CB, _NWT), jnp.int32),
            pltpu.VMEM((_CB, _NPT), jnp.int32),
            pltpu.VMEM((_CB, _NPT), jnp.int32),
            pltpu.SemaphoreType.DMA,
            pltpu.SemaphoreType.DMA,
            pltpu.SemaphoreType.DMA,
            pltpu.SemaphoreType.DMA,
        ],
    )
    return f(addresses, packed)


def kernel(input_bits, connections, memory):
    bits_u8 = input_bits.astype(jnp.uint8)
    addresses = _addresses(bits_u8, connections)
    packed = _pack_memory(memory)
    out = _lookup(addresses, packed)
    return out.astype(jnp.bool_)
